# baseline (speedup 1.0000x reference)
import functools

import jax
import jax.numpy as jnp
from jax import lax
from jax.experimental import pallas as pl
from jax.experimental.pallas import tpu as pltpu
from jax.experimental.pallas import tpu_sc as plsc

D = 768
NE = 64
T = 2048
BM = 128
_BSHIFT = BM.bit_length() - 1
SBM = 512
TP = T + NE * 8

_NC = 2
_NS = 16
_NW = _NC * _NS
_BPW = T // _NW


def _router_body(x_ref, rw_ref, bias_ref, wfc_ref, wproj_ref,
                 eid_ref, gate_ref, sh_ref):
    x = x_ref[...]
    rw = rw_ref[...]
    logits = lax.dot_general(x, rw, (((1,), (1,)), ((), ())),
                             preferred_element_type=jnp.float32)
    scores = jax.nn.sigmoid(logits)
    sel = scores + bias_ref[...]
    m = jnp.max(sel, axis=1, keepdims=True)
    iota = lax.broadcasted_iota(jnp.int32, sel.shape, 1)
    idx = jnp.min(jnp.where(sel == m, iota, NE), axis=1, keepdims=True)
    s = jnp.max(jnp.where(iota == idx, scores, -jnp.inf), axis=1, keepdims=True)
    eid_ref[...] = idx
    gate_ref[...] = s / (s + 1e-20)
    h = lax.dot_general(x, wfc_ref[...], (((1,), (1,)), ((), ())),
                        preferred_element_type=jnp.float32)
    h = jnp.square(jnp.maximum(h, 0.0))
    sh_ref[...] = lax.dot_general(h, wproj_ref[...], (((1,), (1,)), ((), ())),
                                  preferred_element_type=jnp.float32)


def _router_shared(x_flat, router_weight, balance_bias, w_fc_s, w_proj_s):
    return pl.pallas_call(
        _router_body,
        grid=(T // SBM,),
        in_specs=[
            pl.BlockSpec((SBM, D), lambda i: (i, 0)),
            pl.BlockSpec((NE, D), lambda i: (0, 0)),
            pl.BlockSpec((1, NE), lambda i: (0, 0)),
            pl.BlockSpec((D, D), lambda i: (0, 0)),
            pl.BlockSpec((D, D), lambda i: (0, 0)),
        ],
        out_specs=[
            pl.BlockSpec((SBM, 1), lambda i: (i, 0)),
            pl.BlockSpec((SBM, 1), lambda i: (i, 0)),
            pl.BlockSpec((SBM, D), lambda i: (i, 0)),
        ],
        out_shape=[
            jax.ShapeDtypeStruct((T, 1), jnp.int32),
            jax.ShapeDtypeStruct((T, 1), jnp.float32),
            jax.ShapeDtypeStruct((T, D), jnp.float32),
        ],
    )(x_flat, router_weight, balance_bias.reshape(1, NE), w_fc_s, w_proj_s)


NSLOTS = 96


def _sc_dispatch_meta(eid, x_flat):
    mesh = plsc.VectorSubcoreMesh(core_axis_name="c", subcore_axis_name="s")
    i32 = jnp.int32
    nch = NE // 16

    @functools.partial(
        pl.kernel,
        mesh=mesh,
        compiler_params=pltpu.CompilerParams(needs_layout_passes=False),
        out_type=[
            jax.ShapeDtypeStruct((T,), i32),
            jax.ShapeDtypeStruct((_NW, NE), i32),
        ],
        scratch_types=[
            pltpu.VMEM((_BPW,), i32),
            pltpu.VMEM((NE,), i32),
            pltpu.VMEM((_BPW,), i32),
        ],
    )
    def k1(eid_hbm, rloc_hbm, hists_hbm, eid_v, hist_v, rloc_v):
        wid = lax.axis_index("s") * _NC + lax.axis_index("c")
        base = wid * _BPW
        pltpu.sync_copy(eid_hbm.at[pl.ds(base, _BPW)], eid_v)
        lane = lax.iota(i32, 16)
        ones = jnp.ones((16,), i32)
        lane0 = lane == 0
        for cc in range(nch):
            hist_v[pl.ds(16 * cc, 16)] = jnp.zeros((16,), i32)
        for c in range(_BPW // 16):
            tv = eid_v[pl.ds(16 * c, 16)]
            rl = jnp.zeros((16,), i32)
            for j in range(16):
                et = jnp.broadcast_to(tv[j], (16,))
                rt = plsc.load_gather(hist_v, [et])
                rl = jnp.where(lane == j, rt, rl)
                plsc.addupdate_scatter(hist_v, [et], ones, mask=lane0)
            rloc_v[pl.ds(16 * c, 16)] = rl
        pltpu.sync_copy(rloc_v, rloc_hbm.at[pl.ds(base, _BPW)])
        pltpu.sync_copy(hist_v, hists_hbm.at[wid])

    @functools.partial(
        pl.kernel,
        mesh=mesh,
        compiler_params=pltpu.CompilerParams(needs_layout_passes=False),
        out_type=[
            jax.ShapeDtypeStruct((T,), i32),
            jax.ShapeDtypeStruct((TP, D), jnp.float32),
            jax.ShapeDtypeStruct((NE,), i32),
            jax.ShapeDtypeStruct((NE,), i32),
            jax.ShapeDtypeStruct((NSLOTS,), i32),
            jax.ShapeDtypeStruct((NSLOTS,), i32),
        ],
        scratch_types=[
            pltpu.VMEM((_BPW,), i32),
            pltpu.VMEM((_BPW,), i32),
            pltpu.VMEM((_NW, NE), i32),
            pltpu.VMEM((NE,), i32),
            pltpu.VMEM((_BPW,), i32),
            pltpu.VMEM((_BPW, D), jnp.float32),
            pltpu.VMEM((NE,), i32),
            pltpu.VMEM((NE,), i32),
            pltpu.VMEM((NE,), i32),
            pltpu.VMEM((NSLOTS,), i32),
            pltpu.VMEM((NSLOTS,), i32),
            pltpu.SemaphoreType.DMA,
        ],
    )
    def k2(eid_hbm, rloc_hbm, hists_hbm, x_hbm,
           rank_hbm, xs_hbm, sizes_hbm, offs_hbm, se_hbm, sb_hbm,
           eid_v, rloc_v, hists_v, base_v, rank_v, rows_v, sizes_v, offs_v,
           cumx_v, se_v, sb_v, sem):
        wid = lax.axis_index("s") * _NC + lax.axis_index("c")
        base = wid * _BPW
        xcopy = pltpu.async_copy(x_hbm.at[pl.ds(base, _BPW)], rows_v, sem)
        pltpu.sync_copy(eid_hbm.at[pl.ds(base, _BPW)], eid_v)
        pltpu.sync_copy(rloc_hbm.at[pl.ds(base, _BPW)], rloc_v)
        pltpu.sync_copy(hists_hbm, hists_v)
        lane = lax.iota(i32, 16)
        zero = jnp.zeros((16,), i32)

        tot = [zero] * nch
        bef = [zero] * nch
        for t in range(_NW):
            tlt = t < wid
            for cc in range(nch):
                hv = hists_v[t, pl.ds(16 * cc, 16)]
                tot[cc] = tot[cc] + hv
                bef[cc] = bef[cc] + jnp.where(tlt, hv, zero)

        run = jnp.zeros((), i32)
        offs = []
        spads = []
        for cc in range(nch):
            spad = (tot[cc] + 7) & (-8)
            oc = jnp.zeros((16,), i32)
            for j in range(16):
                oc = jnp.where(lane == j, run, oc)
                run = run + spad[j]
            offs.append(oc)
            spads.append(spad)
        for cc in range(nch):
            base_v[pl.ds(16 * cc, 16)] = offs[cc] + bef[cc]

        for c in range(_BPW // 16):
            tv = eid_v[pl.ds(16 * c, 16)]
            g = plsc.load_gather(base_v, [tv])
            rank_v[pl.ds(16 * c, 16)] = rloc_v[pl.ds(16 * c, 16)] + g
        pltpu.sync_copy(rank_v, rank_hbm.at[pl.ds(base, _BPW)])
        xcopy.wait()
        pltpu.async_copy(rows_v, xs_hbm.at[rank_v], sem).wait()

        @pl.when(wid == 0)
        def _():
            for cc in range(nch):
                sizes_v[pl.ds(16 * cc, 16)] = tot[cc]
                offs_v[pl.ds(16 * cc, 16)] = offs[cc]
            pltpu.sync_copy(sizes_v, sizes_hbm)
            pltpu.sync_copy(offs_v, offs_hbm)
            trun = jnp.zeros((), i32)
            tincl = []
            for cc in range(nch):
                nb = (spads[cc] + (BM - 1)) >> _BSHIFT
                xc = jnp.zeros((16,), i32)
                for j in range(16):
                    xc = jnp.where(lane == j, trun, xc)
                    trun = trun + nb[j]
                    tincl.append(trun)
                cumx_v[pl.ds(16 * cc, 16)] = xc
            for s in range(NSLOTS // 16):
                sv = lane + 16 * s
                acc = zero
                for e in range(NE):
                    acc = acc + jnp.where(sv >= tincl[e], 1, 0)
                ecl = jnp.minimum(acc, NE - 1)
                se_v[pl.ds(16 * s, 16)] = ecl
                sb_v[pl.ds(16 * s, 16)] = sv - plsc.load_gather(cumx_v, [ecl])
            pltpu.sync_copy(se_v, se_hbm)
            pltpu.sync_copy(sb_v, sb_hbm)

    rloc, hists = k1(eid)
    return k2(eid, rloc, hists, x_flat)


def _sc_permute_gather(ys, rank):
    mesh = plsc.VectorSubcoreMesh(core_axis_name="c", subcore_axis_name="s")

    @functools.partial(
        pl.kernel,
        mesh=mesh,
        out_type=jax.ShapeDtypeStruct((T, D), jnp.float32),
        scratch_types=[
            pltpu.VMEM((_BPW,), jnp.int32),
            pltpu.VMEM((_BPW, D), jnp.float32),
            pltpu.SemaphoreType.DMA,
        ],
    )
    def k(ys_hbm, r_hbm, out_hbm, idx_v, rows_v, sem):
        wid = lax.axis_index("s") * _NC + lax.axis_index("c")
        base = wid * _BPW
        pltpu.sync_copy(r_hbm.at[pl.ds(base, _BPW)], idx_v)
        pltpu.async_copy(ys_hbm.at[idx_v], rows_v, sem).wait()
        pltpu.sync_copy(rows_v, out_hbm.at[pl.ds(base, _BPW)])

    return k(ys, rank)


def _gmm_body(se_ref, sb_ref, off_ref, size_ref,
              xs_ref, wfc_ref, wproj_ref, out_ref):
    i = pl.program_id(0)
    e = se_ref[i]
    b = sb_ref[i]
    size = size_ref[e]

    @pl.when(b * BM < size)
    def _():
        off = off_ref[e]
        start = pl.multiple_of(jnp.minimum(off + b * BM, TP - BM), 8)
        xb = xs_ref[pl.ds(start, BM), :]
        h = lax.dot_general(xb, wfc_ref[0], (((1,), (1,)), ((), ())),
                            preferred_element_type=jnp.float32)
        h = jnp.square(jnp.maximum(h, 0.0))
        yb = lax.dot_general(h, wproj_ref[0], (((1,), (1,)), ((), ())),
                             preferred_element_type=jnp.float32)
        rows = start + lax.broadcasted_iota(jnp.int32, (BM, 1), 0)
        mask = (rows >= off) & (rows < off + size)
        out_ref[pl.ds(start, BM), :] = jnp.where(
            mask, yb, out_ref[pl.ds(start, BM), :])


def _gmm(slot_e, slot_b, offsets, sizes, xs, w_fc_experts, w_proj_experts):
    grid_spec = pltpu.PrefetchScalarGridSpec(
        num_scalar_prefetch=4,
        grid=(NSLOTS,),
        in_specs=[
            pl.BlockSpec((TP, D), lambda i, se, sb, off, sz: (0, 0)),
            pl.BlockSpec((1, D, D), lambda i, se, sb, off, sz: (se[i], 0, 0)),
            pl.BlockSpec((1, D, D), lambda i, se, sb, off, sz: (se[i], 0, 0)),
        ],
        out_specs=pl.BlockSpec((TP, D), lambda i, se, sb, off, sz: (0, 0)),
    )
    return pl.pallas_call(
        _gmm_body,
        grid_spec=grid_spec,
        out_shape=jax.ShapeDtypeStruct((TP, D), jnp.float32),
    )(slot_e, slot_b, offsets, sizes, xs, w_fc_experts, w_proj_experts)


def _add_body(sh_ref, routed_ref, gate_ref, out_ref):
    out_ref[...] = sh_ref[...] + gate_ref[...] * routed_ref[...]


def _final_add(shared, routed, gate):
    return pl.pallas_call(
        _add_body,
        grid=(T // SBM,),
        in_specs=[
            pl.BlockSpec((SBM, D), lambda i: (i, 0)),
            pl.BlockSpec((SBM, D), lambda i: (i, 0)),
            pl.BlockSpec((SBM, 1), lambda i: (i, 0)),
        ],
        out_specs=pl.BlockSpec((SBM, D), lambda i: (i, 0)),
        out_shape=jax.ShapeDtypeStruct((T, D), jnp.float32),
    )(shared, routed, gate)


def kernel(x, w_fc_shared, w_proj_shared, w_fc_experts, w_proj_experts,
           router_weight, balance_bias):
    B, Tn, C = x.shape
    x_flat = x.reshape(T, D)

    eid2, gate2, shared = _router_shared(
        x_flat, router_weight, balance_bias, w_fc_shared, w_proj_shared)
    eid = eid2[:, 0]

    rank, xs, sizes, offsets_pad, slot_e, slot_b = _sc_dispatch_meta(
        eid, x_flat)
    ys = _gmm(slot_e, slot_b, offsets_pad, sizes, xs,
              w_fc_experts, w_proj_experts)
    routed = _sc_permute_gather(ys, rank)
    out = _final_add(shared, routed, gate2)
    return out.reshape(B, Tn, C)


# speedup vs baseline: 1.0078x; 1.0078x over previous
import functools

import jax
import jax.numpy as jnp
from jax import lax
from jax.experimental import pallas as pl
from jax.experimental.pallas import tpu as pltpu
from jax.experimental.pallas import tpu_sc as plsc

D = 768
NE = 64
T = 2048
BM = 128
_BSHIFT = BM.bit_length() - 1
SBM = 512
TP = T + NE * 8

_NC = 2
_NS = 16
_NW = _NC * _NS
_BPW = T // _NW


def _router_body(x_ref, rw_ref, bias_ref, eid_ref, gate_ref):
    x = x_ref[...]
    rw = rw_ref[...]
    logits = lax.dot_general(x, rw, (((1,), (1,)), ((), ())),
                             preferred_element_type=jnp.float32)
    scores = jax.nn.sigmoid(logits)
    sel = scores + bias_ref[...]
    m = jnp.max(sel, axis=1, keepdims=True)
    iota = lax.broadcasted_iota(jnp.int32, sel.shape, 1)
    idx = jnp.min(jnp.where(sel == m, iota, NE), axis=1, keepdims=True)
    s = jnp.max(jnp.where(iota == idx, scores, -jnp.inf), axis=1, keepdims=True)
    eid_ref[...] = idx
    gate_ref[...] = s / (s + 1e-20)


def _router(x_flat, router_weight, balance_bias):
    return pl.pallas_call(
        _router_body,
        out_shape=[
            jax.ShapeDtypeStruct((T, 1), jnp.int32),
            jax.ShapeDtypeStruct((T, 1), jnp.float32),
        ],
    )(x_flat, router_weight, balance_bias.reshape(1, NE))


def _shared_body(x_ref, wfc_ref, wproj_ref, out_ref):
    xb = x_ref[...]
    h = lax.dot_general(xb, wfc_ref[...], (((1,), (1,)), ((), ())),
                        preferred_element_type=jnp.float32)
    h = jnp.square(jnp.maximum(h, 0.0))
    out_ref[...] = lax.dot_general(h, wproj_ref[...], (((1,), (1,)), ((), ())),
                                   preferred_element_type=jnp.float32)


def _shared_mlp(x_flat, w_fc_shared, w_proj_shared):
    return pl.pallas_call(
        _shared_body,
        grid=(T // SBM,),
        in_specs=[
            pl.BlockSpec((SBM, D), lambda i: (i, 0)),
            pl.BlockSpec((D, D), lambda i: (0, 0)),
            pl.BlockSpec((D, D), lambda i: (0, 0)),
        ],
        out_specs=pl.BlockSpec((SBM, D), lambda i: (i, 0)),
        out_shape=jax.ShapeDtypeStruct((T, D), jnp.float32),
    )(x_flat, w_fc_shared, w_proj_shared)


NSLOTS = 96


def _sc_dispatch_meta(eid, x_flat):
    mesh = plsc.VectorSubcoreMesh(core_axis_name="c", subcore_axis_name="s")
    i32 = jnp.int32
    nch = NE // 16

    @functools.partial(
        pl.kernel,
        mesh=mesh,
        compiler_params=pltpu.CompilerParams(needs_layout_passes=False),
        out_type=[
            jax.ShapeDtypeStruct((T,), i32),
            jax.ShapeDtypeStruct((_NW, NE), i32),
        ],
        scratch_types=[
            pltpu.VMEM((_BPW,), i32),
            pltpu.VMEM((NE,), i32),
            pltpu.VMEM((_BPW,), i32),
        ],
    )
    def k1(eid_hbm, rloc_hbm, hists_hbm, eid_v, hist_v, rloc_v):
        wid = lax.axis_index("s") * _NC + lax.axis_index("c")
        base = wid * _BPW
        pltpu.sync_copy(eid_hbm.at[pl.ds(base, _BPW)], eid_v)
        lane = lax.iota(i32, 16)
        ones = jnp.ones((16,), i32)
        lane0 = lane == 0
        for cc in range(nch):
            hist_v[pl.ds(16 * cc, 16)] = jnp.zeros((16,), i32)
        for c in range(_BPW // 16):
            tv = eid_v[pl.ds(16 * c, 16)]
            rl = jnp.zeros((16,), i32)
            for j in range(16):
                et = jnp.broadcast_to(tv[j], (16,))
                rt = plsc.load_gather(hist_v, [et])
                rl = jnp.where(lane == j, rt, rl)
                plsc.addupdate_scatter(hist_v, [et], ones, mask=lane0)
            rloc_v[pl.ds(16 * c, 16)] = rl
        pltpu.sync_copy(rloc_v, rloc_hbm.at[pl.ds(base, _BPW)])
        pltpu.sync_copy(hist_v, hists_hbm.at[wid])

    @functools.partial(
        pl.kernel,
        mesh=mesh,
        compiler_params=pltpu.CompilerParams(needs_layout_passes=False),
        out_type=[
            jax.ShapeDtypeStruct((T,), i32),
            jax.ShapeDtypeStruct((TP, D), jnp.float32),
            jax.ShapeDtypeStruct((NE,), i32),
            jax.ShapeDtypeStruct((NE,), i32),
            jax.ShapeDtypeStruct((NSLOTS,), i32),
            jax.ShapeDtypeStruct((NSLOTS,), i32),
        ],
        scratch_types=[
            pltpu.VMEM((_BPW,), i32),
            pltpu.VMEM((_BPW,), i32),
            pltpu.VMEM((_NW, NE), i32),
            pltpu.VMEM((NE,), i32),
            pltpu.VMEM((_BPW,), i32),
            pltpu.VMEM((_BPW, D), jnp.float32),
            pltpu.VMEM((NE,), i32),
            pltpu.VMEM((NE,), i32),
            pltpu.VMEM((NE,), i32),
            pltpu.VMEM((NSLOTS,), i32),
            pltpu.VMEM((NSLOTS,), i32),
            pltpu.SemaphoreType.DMA,
        ],
    )
    def k2(eid_hbm, rloc_hbm, hists_hbm, x_hbm,
           rank_hbm, xs_hbm, sizes_hbm, offs_hbm, se_hbm, sb_hbm,
           eid_v, rloc_v, hists_v, base_v, rank_v, rows_v, sizes_v, offs_v,
           cumx_v, se_v, sb_v, sem):
        wid = lax.axis_index("s") * _NC + lax.axis_index("c")
        base = wid * _BPW
        xcopy = pltpu.async_copy(x_hbm.at[pl.ds(base, _BPW)], rows_v, sem)
        pltpu.sync_copy(eid_hbm.at[pl.ds(base, _BPW)], eid_v)
        pltpu.sync_copy(rloc_hbm.at[pl.ds(base, _BPW)], rloc_v)
        pltpu.sync_copy(hists_hbm, hists_v)
        lane = lax.iota(i32, 16)
        zero = jnp.zeros((16,), i32)

        tot = [zero] * nch
        bef = [zero] * nch
        for t in range(_NW):
            tlt = t < wid
            for cc in range(nch):
                hv = hists_v[t, pl.ds(16 * cc, 16)]
                tot[cc] = tot[cc] + hv
                bef[cc] = bef[cc] + jnp.where(tlt, hv, zero)

        run = jnp.zeros((), i32)
        offs = []
        spads = []
        for cc in range(nch):
            spad = (tot[cc] + 7) & (-8)
            oc = jnp.zeros((16,), i32)
            for j in range(16):
                oc = jnp.where(lane == j, run, oc)
                run = run + spad[j]
            offs.append(oc)
            spads.append(spad)
        for cc in range(nch):
            base_v[pl.ds(16 * cc, 16)] = offs[cc] + bef[cc]

        for c in range(_BPW // 16):
            tv = eid_v[pl.ds(16 * c, 16)]
            g = plsc.load_gather(base_v, [tv])
            rank_v[pl.ds(16 * c, 16)] = rloc_v[pl.ds(16 * c, 16)] + g
        pltpu.sync_copy(rank_v, rank_hbm.at[pl.ds(base, _BPW)])
        xcopy.wait()
        pltpu.async_copy(rows_v, xs_hbm.at[rank_v], sem).wait()

        @pl.when(wid == 0)
        def _():
            for cc in range(nch):
                sizes_v[pl.ds(16 * cc, 16)] = tot[cc]
                offs_v[pl.ds(16 * cc, 16)] = offs[cc]
            pltpu.sync_copy(sizes_v, sizes_hbm)
            pltpu.sync_copy(offs_v, offs_hbm)
            trun = jnp.zeros((), i32)
            tincl = []
            for cc in range(nch):
                nb = (spads[cc] + (BM - 1)) >> _BSHIFT
                xc = jnp.zeros((16,), i32)
                for j in range(16):
                    xc = jnp.where(lane == j, trun, xc)
                    trun = trun + nb[j]
                    tincl.append(trun)
                cumx_v[pl.ds(16 * cc, 16)] = xc
            for s in range(NSLOTS // 16):
                sv = lane + 16 * s
                acc = zero
                for e in range(NE):
                    acc = acc + jnp.where(sv >= tincl[e], 1, 0)
                ecl = jnp.minimum(acc, NE - 1)
                se_v[pl.ds(16 * s, 16)] = ecl
                sb_v[pl.ds(16 * s, 16)] = sv - plsc.load_gather(cumx_v, [ecl])
            pltpu.sync_copy(se_v, se_hbm)
            pltpu.sync_copy(sb_v, sb_hbm)

    rloc, hists = k1(eid)
    return k2(eid, rloc, hists, x_flat)


def _sc_permute_gather(ys, rank):
    mesh = plsc.VectorSubcoreMesh(core_axis_name="c", subcore_axis_name="s")

    @functools.partial(
        pl.kernel,
        mesh=mesh,
        out_type=jax.ShapeDtypeStruct((T, D), jnp.float32),
        scratch_types=[
            pltpu.VMEM((_BPW,), jnp.int32),
            pltpu.VMEM((_BPW, D), jnp.float32),
            pltpu.SemaphoreType.DMA,
        ],
    )
    def k(ys_hbm, r_hbm, out_hbm, idx_v, rows_v, sem):
        wid = lax.axis_index("s") * _NC + lax.axis_index("c")
        base = wid * _BPW
        pltpu.sync_copy(r_hbm.at[pl.ds(base, _BPW)], idx_v)
        pltpu.async_copy(ys_hbm.at[idx_v], rows_v, sem).wait()
        pltpu.sync_copy(rows_v, out_hbm.at[pl.ds(base, _BPW)])

    return k(ys, rank)


def _gmm_body(se_ref, sb_ref, off_ref, size_ref,
              xs_ref, wfc_ref, wproj_ref, out_ref):
    i = pl.program_id(0)
    e = se_ref[i]
    b = sb_ref[i]
    size = size_ref[e]

    @pl.when(b * BM < size)
    def _():
        off = off_ref[e]
        start = pl.multiple_of(jnp.minimum(off + b * BM, TP - BM), 8)
        xb = xs_ref[pl.ds(start, BM), :]
        h = lax.dot_general(xb, wfc_ref[0], (((1,), (1,)), ((), ())),
                            preferred_element_type=jnp.float32)
        h = jnp.square(jnp.maximum(h, 0.0))
        yb = lax.dot_general(h, wproj_ref[0], (((1,), (1,)), ((), ())),
                             preferred_element_type=jnp.float32)
        rows = start + lax.broadcasted_iota(jnp.int32, (BM, 1), 0)
        mask = (rows >= off) & (rows < off + size)
        out_ref[pl.ds(start, BM), :] = jnp.where(
            mask, yb, out_ref[pl.ds(start, BM), :])


def _gmm(slot_e, slot_b, offsets, sizes, xs, w_fc_experts, w_proj_experts):
    grid_spec = pltpu.PrefetchScalarGridSpec(
        num_scalar_prefetch=4,
        grid=(NSLOTS,),
        in_specs=[
            pl.BlockSpec((TP, D), lambda i, se, sb, off, sz: (0, 0)),
            pl.BlockSpec((1, D, D), lambda i, se, sb, off, sz: (se[i], 0, 0)),
            pl.BlockSpec((1, D, D), lambda i, se, sb, off, sz: (se[i], 0, 0)),
        ],
        out_specs=pl.BlockSpec((TP, D), lambda i, se, sb, off, sz: (0, 0)),
    )
    return pl.pallas_call(
        _gmm_body,
        grid_spec=grid_spec,
        out_shape=jax.ShapeDtypeStruct((TP, D), jnp.float32),
    )(slot_e, slot_b, offsets, sizes, xs, w_fc_experts, w_proj_experts)


def _add_body(sh_ref, routed_ref, gate_ref, out_ref):
    out_ref[...] = sh_ref[...] + gate_ref[...] * routed_ref[...]


def _final_add(shared, routed, gate):
    return pl.pallas_call(
        _add_body,
        grid=(T // SBM,),
        in_specs=[
            pl.BlockSpec((SBM, D), lambda i: (i, 0)),
            pl.BlockSpec((SBM, D), lambda i: (i, 0)),
            pl.BlockSpec((SBM, 1), lambda i: (i, 0)),
        ],
        out_specs=pl.BlockSpec((SBM, D), lambda i: (i, 0)),
        out_shape=jax.ShapeDtypeStruct((T, D), jnp.float32),
    )(shared, routed, gate)


def kernel(x, w_fc_shared, w_proj_shared, w_fc_experts, w_proj_experts,
           router_weight, balance_bias):
    B, Tn, C = x.shape
    x_flat = x.reshape(T, D)

    eid2, gate2 = _router(x_flat, router_weight, balance_bias)
    eid = eid2[:, 0]
    shared = _shared_mlp(x_flat, w_fc_shared, w_proj_shared)

    rank, xs, sizes, offsets_pad, slot_e, slot_b = _sc_dispatch_meta(
        eid, x_flat)
    ys = _gmm(slot_e, slot_b, offsets_pad, sizes, xs,
              w_fc_experts, w_proj_experts)
    routed = _sc_permute_gather(ys, rank)
    out = _final_add(shared, routed, gate2)
    return out.reshape(B, Tn, C)
